# baseline (speedup 1.0000x reference)
import functools

import jax
import jax.numpy as jnp
from jax import lax
from jax.experimental import pallas as pl
from jax.experimental.pallas import tpu as pltpu
from jax.experimental.pallas import tpu_sc as plsc

CHUNK = 128
CPB = 20


@functools.lru_cache(maxsize=None)
def _make_lookup(n_idx, dim, num_cores, num_subcores):
    nw = num_cores * num_subcores
    chunks_pw = n_idx // (nw * CHUNK)
    bpw = chunks_pw // CPB
    blk = CPB * CHUNK
    mesh = plsc.VectorSubcoreMesh(core_axis_name="c", subcore_axis_name="s")

    @functools.partial(
        pl.kernel,
        mesh=mesh,
        out_type=jax.ShapeDtypeStruct((n_idx, dim), jnp.int32),
        scratch_types=[
            pltpu.VMEM((chunks_pw, CHUNK), jnp.int32),
            pltpu.VMEM((blk, dim), jnp.int32),
            pltpu.SemaphoreType.DMA,
        ],
        compiler_params=pltpu.CompilerParams(use_tc_tiling_on_sc=False),
    )
    def lookup(tab_hbm, idx_hbm, out_hbm, idx_v, rows_v, gsem):
        wid = lax.axis_index("s") * num_cores + lax.axis_index("c")
        cbase = wid * chunks_pw
        pltpu.sync_copy(idx_hbm.at[pl.ds(cbase, chunks_pw)], idx_v)

        def block(b, carry):
            waits = []
            for j in range(CPB):
                waits.append(
                    pltpu.async_copy(
                        tab_hbm.at[idx_v.at[b * CPB + j]],
                        rows_v.at[pl.ds(j * CHUNK, CHUNK)],
                        gsem,
                    )
                )
            for w in waits:
                w.wait()
            pltpu.sync_copy(
                rows_v,
                out_hbm.at[pl.ds((cbase + b * CPB) * CHUNK, blk)],
            )
            return carry

        lax.fori_loop(0, bpw, block, 0)

    return lookup


def kernel(input, weight):
    b, h = input.shape
    v, dim = weight.shape
    n = b * h
    words = dim // 2
    idx = input.reshape(n // CHUNK, CHUNK).astype(jnp.int32)
    w32 = jax.lax.bitcast_convert_type(
        weight.reshape(v, words, 2), jnp.int32
    )
    info = plsc.get_sparse_core_info()
    out32 = _make_lookup(n, words, info.num_cores, info.num_subcores)(w32, idx)
    out = jax.lax.bitcast_convert_type(out32, jnp.bfloat16)
    return out.reshape(b, h, dim)


# speedup vs baseline: 1.2916x; 1.2916x over previous
import functools

import jax
import jax.numpy as jnp
from jax import lax
from jax.experimental import pallas as pl
from jax.experimental.pallas import tpu as pltpu
from jax.experimental.pallas import tpu_sc as plsc

CONV_BLK = 625
CONV_UNROLL = 5
GPB = 16


@functools.lru_cache(maxsize=None)
def _make_lookup(b, h, v, dim, num_cores, num_subcores):
    words = dim // 2
    nw = num_cores * num_subcores
    rows_pw = b // nw
    n_pw = rows_pw * h
    gblocks = rows_pw // GPB
    blk_out = GPB * h
    rows_pt = v // num_subcores
    cblocks = rows_pt // CONV_BLK
    mesh = plsc.VectorSubcoreMesh(core_axis_name="c", subcore_axis_name="s")

    @functools.partial(
        pl.kernel,
        mesh=mesh,
        out_type=(
            jax.ShapeDtypeStruct((b * h, dim), jnp.bfloat16),
            jax.ShapeDtypeStruct((v, words), jnp.int32),
            jax.ShapeDtypeStruct((v, words), jnp.int32),
        ),
        scratch_types=[
            pltpu.VMEM((CONV_BLK, dim), jnp.bfloat16),
            pltpu.VMEM((CONV_BLK, dim), jnp.bfloat16),
            pltpu.VMEM((CONV_BLK, words), jnp.int32),
            pltpu.VMEM((CONV_BLK, words), jnp.int32),
            pltpu.VMEM((rows_pw, h), jnp.int32),
            pltpu.VMEM((blk_out, words), jnp.int32),
            pltpu.VMEM((blk_out, dim), jnp.bfloat16),
            pltpu.SemaphoreType.DMA,
            pltpu.SemaphoreType.DMA,
            pltpu.SemaphoreType.DMA,
            pltpu.SemaphoreType.DMA,
            pltpu.SemaphoreType.DMA,
        ],
        compiler_params=pltpu.CompilerParams(
            use_tc_tiling_on_sc=False, needs_layout_passes=False
        ),
    )
    def lookup(w_hbm, idx_hbm, out_hbm, tab0_hbm, tab1_hbm,
               vb0, vb1, vi0, vi1, idxv, rows_v, vbf,
               cin0, cin1, cout0, cout1, gsem):
        c = lax.axis_index("c")
        s = lax.axis_index("s")
        wid = s * num_cores + c
        vbs = [vb0, vb1]
        vis = [vi0, vi1]
        cins = [cin0, cin1]
        couts = [cout0, cout1]

        def convert_phase(tab_hbm):
            base = s * rows_pt

            def w_slice(blkno):
                return w_hbm.at[pl.ds(base + blkno * CONV_BLK, CONV_BLK)]

            def t_slice(blkno):
                return tab_hbm.at[pl.ds(base + blkno * CONV_BLK, CONV_BLK)]

            pltpu.async_copy(w_slice(0), vb0, cin0)
            pltpu.async_copy(w_slice(1), vb1, cin1)

            def cbody(g, carry):
                for k in (0, 1):
                    blkno = 2 * g + k
                    pltpu.make_async_copy(w_slice(0), vbs[k], cins[k]).wait()

                    @pl.when(g >= 1)
                    def _():
                        pltpu.make_async_copy(vis[k], t_slice(0), couts[k]).wait()

                    def rbody(r, carry2):
                        base_r = r * CONV_UNROLL
                        for u in range(CONV_UNROLL):
                            vis[k][base_r + u] = plsc.bitcast(
                                vbs[k][base_r + u], jnp.int32
                            )
                        return carry2

                    lax.fori_loop(0, CONV_BLK // CONV_UNROLL, rbody, 0)
                    pltpu.async_copy(vis[k], t_slice(blkno), couts[k])

                    @pl.when(blkno + 2 < cblocks)
                    def _():
                        pltpu.async_copy(w_slice(blkno + 2), vbs[k], cins[k])

                return carry

            lax.fori_loop(0, cblocks // 2, cbody, 0)
            pltpu.make_async_copy(vi0, t_slice(0), cout0).wait()
            pltpu.make_async_copy(vi1, t_slice(0), cout1).wait()

        def gather_phase(tab_hbm):
            pltpu.sync_copy(idx_hbm.at[pl.ds(wid * rows_pw, rows_pw)], idxv)

            def gbody(gb, carry):
                waits = []
                for j in range(GPB):
                    waits.append(
                        pltpu.async_copy(
                            tab_hbm.at[idxv.at[gb * GPB + j]],
                            rows_v.at[pl.ds(j * h, h)],
                            gsem,
                        )
                    )
                for w in waits:
                    w.wait()

                def obody(r, carry2):
                    base_r = r * 8
                    for u in range(8):
                        vbf[base_r + u] = plsc.bitcast(
                            rows_v[base_r + u], jnp.bfloat16
                        )
                    return carry2

                lax.fori_loop(0, blk_out // 8, obody, 0)
                pltpu.sync_copy(
                    vbf,
                    out_hbm.at[pl.ds((wid * gblocks + gb) * blk_out, blk_out)],
                )
                return carry

            lax.fori_loop(0, gblocks, gbody, 0)

        @pl.when(c == 0)
        def _():
            convert_phase(tab0_hbm)

        @pl.when(c == 1)
        def _():
            convert_phase(tab1_hbm)

        plsc.subcore_barrier()

        @pl.when(c == 0)
        def _():
            gather_phase(tab0_hbm)

        @pl.when(c == 1)
        def _():
            gather_phase(tab1_hbm)

    return lookup


def kernel(input, weight):
    b, h = input.shape
    v, dim = weight.shape
    info = plsc.get_sparse_core_info()
    out, _, _ = _make_lookup(b, h, v, dim, info.num_cores, info.num_subcores)(
        weight, input.astype(jnp.int32)
    )
    return out.reshape(b, h, dim)


# speedup vs baseline: 2.0358x; 1.5762x over previous
import functools

import jax
import jax.numpy as jnp
from jax import lax
from jax.experimental import pallas as pl
from jax.experimental.pallas import tpu as pltpu
from jax.experimental.pallas import tpu_sc as plsc

CONV_BLK = 625
CONV_UNROLL = 5
GPB = 16


@functools.lru_cache(maxsize=None)
def _make_lookup(b, h, v, dim, num_cores, num_subcores):
    words = dim // 2
    nw = num_cores * num_subcores
    rows_pw = b // nw
    n_pw = rows_pw * h
    gblocks = rows_pw // GPB
    blk_out = GPB * h
    rows_pt = v // num_subcores
    cblocks = rows_pt // CONV_BLK
    mesh = plsc.VectorSubcoreMesh(core_axis_name="c", subcore_axis_name="s")

    @functools.partial(
        pl.kernel,
        mesh=mesh,
        out_type=(
            jax.ShapeDtypeStruct((b, h, dim), jnp.bfloat16),
            jax.ShapeDtypeStruct((v, words), jnp.int32),
            jax.ShapeDtypeStruct((v, words), jnp.int32),
        ),
        scratch_types=[
            pltpu.VMEM((CONV_BLK, dim), jnp.bfloat16),
            pltpu.VMEM((CONV_BLK, dim), jnp.bfloat16),
            pltpu.VMEM((CONV_BLK, words), jnp.int32),
            pltpu.VMEM((CONV_BLK, words), jnp.int32),
            pltpu.VMEM((rows_pw, h), jnp.int32),
            pltpu.VMEM((blk_out, words), jnp.int32),
            pltpu.VMEM((GPB, h, dim), jnp.bfloat16),
            pltpu.SemaphoreType.DMA,
            pltpu.SemaphoreType.DMA,
            pltpu.SemaphoreType.DMA,
            pltpu.SemaphoreType.DMA,
            pltpu.SemaphoreType.DMA,
        ],
        compiler_params=pltpu.CompilerParams(
            use_tc_tiling_on_sc=False, needs_layout_passes=False
        ),
    )
    def lookup(w_hbm, idx_hbm, out_hbm, tab0_hbm, tab1_hbm,
               vb0, vb1, vi0, vi1, idxv, rows_v, vbf,
               cin0, cin1, cout0, cout1, gsem):
        c = lax.axis_index("c")
        s = lax.axis_index("s")
        wid = s * num_cores + c
        vbs = [vb0, vb1]
        vis = [vi0, vi1]
        cins = [cin0, cin1]
        couts = [cout0, cout1]

        def convert_phase(tab_hbm):
            base = s * rows_pt

            def w_slice(blkno):
                return w_hbm.at[pl.ds(base + blkno * CONV_BLK, CONV_BLK)]

            def t_slice(blkno):
                return tab_hbm.at[pl.ds(base + blkno * CONV_BLK, CONV_BLK)]

            pltpu.async_copy(w_slice(0), vb0, cin0)
            pltpu.async_copy(w_slice(1), vb1, cin1)

            def cbody(g, carry):
                for k in (0, 1):
                    blkno = 2 * g + k
                    pltpu.make_async_copy(w_slice(0), vbs[k], cins[k]).wait()

                    @pl.when(g >= 1)
                    def _():
                        pltpu.make_async_copy(vis[k], t_slice(0), couts[k]).wait()

                    def rbody(r, carry2):
                        base_r = r * CONV_UNROLL
                        for u in range(CONV_UNROLL):
                            vis[k][base_r + u] = plsc.bitcast(
                                vbs[k][base_r + u], jnp.int32
                            )
                        return carry2

                    lax.fori_loop(0, CONV_BLK // CONV_UNROLL, rbody, 0)
                    pltpu.async_copy(vis[k], t_slice(blkno), couts[k])

                    @pl.when(blkno + 2 < cblocks)
                    def _():
                        pltpu.async_copy(w_slice(blkno + 2), vbs[k], cins[k])

                return carry

            lax.fori_loop(0, cblocks // 2, cbody, 0)
            pltpu.make_async_copy(vi0, t_slice(0), cout0).wait()
            pltpu.make_async_copy(vi1, t_slice(0), cout1).wait()

        def gather_phase(tab_hbm):
            pltpu.sync_copy(idx_hbm.at[pl.ds(wid * rows_pw, rows_pw)], idxv)

            def gbody(gb, carry):
                waits = []
                for j in range(GPB):
                    waits.append(
                        pltpu.async_copy(
                            tab_hbm.at[idxv.at[gb * GPB + j]],
                            rows_v.at[pl.ds(j * h, h)],
                            gsem,
                        )
                    )
                for w in waits:
                    w.wait()

                def obody(i, carry2):
                    base_r = i * h

                    def jbody(j5, carry3):
                        base_j = j5 * 5
                        for u in range(5):
                            vbf[i, base_j + u] = plsc.bitcast(
                                rows_v[base_r + base_j + u], jnp.bfloat16
                            )
                        return carry3

                    lax.fori_loop(0, h // 5, jbody, 0)
                    return carry2

                lax.fori_loop(0, GPB, obody, 0)
                pltpu.sync_copy(
                    vbf,
                    out_hbm.at[pl.ds((wid * gblocks + gb) * GPB, GPB)],
                )
                return carry

            lax.fori_loop(0, gblocks, gbody, 0)

        @pl.when(c == 0)
        def _():
            convert_phase(tab0_hbm)

        @pl.when(c == 1)
        def _():
            convert_phase(tab1_hbm)

        plsc.subcore_barrier()

        @pl.when(c == 0)
        def _():
            gather_phase(tab0_hbm)

        @pl.when(c == 1)
        def _():
            gather_phase(tab1_hbm)

    return lookup


def kernel(input, weight):
    b, h = input.shape
    v, dim = weight.shape
    info = plsc.get_sparse_core_info()
    out, _, _ = _make_lookup(b, h, v, dim, info.num_cores, info.num_subcores)(
        weight, input.astype(jnp.int32)
    )
    return out


# speedup vs baseline: 2.0823x; 1.0228x over previous
import functools

import jax
import jax.numpy as jnp
from jax import lax
from jax.experimental import pallas as pl
from jax.experimental.pallas import tpu as pltpu
from jax.experimental.pallas import tpu_sc as plsc

CONV_BLK = 625
CONV_UNROLL = 5
GPB = 16


@functools.lru_cache(maxsize=None)
def _make_lookup(b, h, v, dim, num_cores, num_subcores):
    words = dim // 2
    nw = num_cores * num_subcores
    rows_pw = b // nw
    n_pw = rows_pw * h
    gblocks = rows_pw // GPB
    blk_out = GPB * h
    rows_pt = v // num_subcores
    cblocks = rows_pt // CONV_BLK
    mesh = plsc.VectorSubcoreMesh(core_axis_name="c", subcore_axis_name="s")

    @functools.partial(
        pl.kernel,
        mesh=mesh,
        out_type=(
            jax.ShapeDtypeStruct((b, h, dim), jnp.bfloat16),
            jax.ShapeDtypeStruct((v, words), jnp.int32),
            jax.ShapeDtypeStruct((v, words), jnp.int32),
        ),
        scratch_types=[
            pltpu.VMEM((CONV_BLK, dim), jnp.bfloat16),
            pltpu.VMEM((CONV_BLK, dim), jnp.bfloat16),
            pltpu.VMEM((CONV_BLK, words), jnp.int32),
            pltpu.VMEM((CONV_BLK, words), jnp.int32),
            pltpu.VMEM((rows_pw, h), jnp.int32),
            pltpu.VMEM((blk_out, words), jnp.int32),
            pltpu.VMEM((blk_out, words), jnp.int32),
            pltpu.VMEM((GPB, h, dim), jnp.bfloat16),
            pltpu.VMEM((GPB, h, dim), jnp.bfloat16),
            pltpu.SemaphoreType.DMA,
            pltpu.SemaphoreType.DMA,
            pltpu.SemaphoreType.DMA,
            pltpu.SemaphoreType.DMA,
            pltpu.SemaphoreType.DMA,
            pltpu.SemaphoreType.DMA,
            pltpu.SemaphoreType.DMA,
            pltpu.SemaphoreType.DMA,
        ],
        compiler_params=pltpu.CompilerParams(
            use_tc_tiling_on_sc=False, needs_layout_passes=False
        ),
    )
    def lookup(w_hbm, idx_hbm, out_hbm, tab0_hbm, tab1_hbm,
               vb0, vb1, vi0, vi1, idxv, rows_v0, rows_v1, vbf0, vbf1,
               cin0, cin1, cout0, cout1, gsem0, gsem1, osem0, osem1):
        c = lax.axis_index("c")
        s = lax.axis_index("s")
        wid = s * num_cores + c
        vbs = [vb0, vb1]
        vis = [vi0, vi1]
        cins = [cin0, cin1]
        couts = [cout0, cout1]

        def convert_phase(tab_hbm):
            base = s * rows_pt

            def w_slice(blkno):
                return w_hbm.at[pl.ds(base + blkno * CONV_BLK, CONV_BLK)]

            def t_slice(blkno):
                return tab_hbm.at[pl.ds(base + blkno * CONV_BLK, CONV_BLK)]

            pltpu.async_copy(w_slice(0), vb0, cin0)
            pltpu.async_copy(w_slice(1), vb1, cin1)

            def cbody(g, carry):
                for k in (0, 1):
                    blkno = 2 * g + k
                    pltpu.make_async_copy(w_slice(0), vbs[k], cins[k]).wait()

                    @pl.when(g >= 1)
                    def _():
                        pltpu.make_async_copy(vis[k], t_slice(0), couts[k]).wait()

                    def rbody(r, carry2):
                        base_r = r * CONV_UNROLL
                        for u in range(CONV_UNROLL):
                            vis[k][base_r + u] = plsc.bitcast(
                                vbs[k][base_r + u], jnp.int32
                            )
                        return carry2

                    lax.fori_loop(0, CONV_BLK // CONV_UNROLL, rbody, 0)
                    pltpu.async_copy(vis[k], t_slice(blkno), couts[k])

                    @pl.when(blkno + 2 < cblocks)
                    def _():
                        pltpu.async_copy(w_slice(blkno + 2), vbs[k], cins[k])

                return carry

            lax.fori_loop(0, cblocks // 2, cbody, 0)
            pltpu.make_async_copy(vi0, t_slice(0), cout0).wait()
            pltpu.make_async_copy(vi1, t_slice(0), cout1).wait()

        def gather_phase(tab_hbm):
            def fire(gb, rows, gsem):
                handles = []
                for j in range(GPB):
                    handles.append(
                        pltpu.async_copy(
                            tab_hbm.at[idxv.at[gb * GPB + j]],
                            rows.at[pl.ds(j * h, h)],
                            gsem,
                        )
                    )
                return handles

            def convert_out(rows, vbuf):
                def obody(i, carry2):
                    base_r = i * h

                    def jbody(j5, carry3):
                        base_j = j5 * 5
                        for u in range(5):
                            vbuf[i, base_j + u] = plsc.bitcast(
                                rows[base_r + base_j + u], jnp.bfloat16
                            )
                        return carry3

                    lax.fori_loop(0, h // 5, jbody, 0)
                    return carry2

                lax.fori_loop(0, GPB, obody, 0)

            rows_bufs = [rows_v0, rows_v1]
            vbf_bufs = [vbf0, vbf1]
            osems = [osem0, osem1]
            gsems = [gsem0, gsem1]

            def gbody(g, carry):
                handles = [fire(2 * g + k, rows_bufs[k], gsems[k]) for k in (0, 1)]
                for k in (0, 1):
                    gb = 2 * g + k
                    for w in handles[k]:
                        w.wait()

                    @pl.when(g >= 1)
                    def _():
                        pltpu.make_async_copy(
                            vbf_bufs[k], out_hbm.at[pl.ds(0, GPB)], osems[k]
                        ).wait()

                    convert_out(rows_bufs[k], vbf_bufs[k])
                    pltpu.async_copy(
                        vbf_bufs[k],
                        out_hbm.at[pl.ds((wid * gblocks + gb) * GPB, GPB)],
                        osems[k],
                    )
                return carry

            lax.fori_loop(0, gblocks // 2, gbody, 0)
            pltpu.make_async_copy(
                vbf0, out_hbm.at[pl.ds(0, GPB)], osem0
            ).wait()
            pltpu.make_async_copy(
                vbf1, out_hbm.at[pl.ds(0, GPB)], osem1
            ).wait()

        @pl.when(c == 0)
        def _():
            convert_phase(tab0_hbm)

        @pl.when(c == 1)
        def _():
            convert_phase(tab1_hbm)

        pltpu.sync_copy(idx_hbm.at[pl.ds(wid * rows_pw, rows_pw)], idxv)
        plsc.subcore_barrier()

        @pl.when(c == 0)
        def _():
            gather_phase(tab0_hbm)

        @pl.when(c == 1)
        def _():
            gather_phase(tab1_hbm)

    return lookup


def kernel(input, weight):
    b, h = input.shape
    v, dim = weight.shape
    info = plsc.get_sparse_core_info()
    out, _, _ = _make_lookup(b, h, v, dim, info.num_cores, info.num_subcores)(
        weight, input.astype(jnp.int32)
    )
    return out


# speedup vs baseline: 2.0840x; 1.0008x over previous
import functools

import jax
import jax.numpy as jnp
from jax import lax
from jax.experimental import pallas as pl
from jax.experimental.pallas import tpu as pltpu
from jax.experimental.pallas import tpu_sc as plsc

CONV_BLK = 625
CONV_UNROLL = 5
GPB = 16


@functools.lru_cache(maxsize=None)
def _make_lookup(b, h, v, dim, num_cores, num_subcores):
    words = dim // 2
    nw = num_cores * num_subcores
    rows_pw = b // nw
    gblocks = rows_pw // GPB
    blk_out = GPB * h
    rows_pt = v // num_subcores
    cblocks = rows_pt // CONV_BLK
    mesh = plsc.VectorSubcoreMesh(core_axis_name="c", subcore_axis_name="s")

    @functools.partial(
        pl.kernel,
        mesh=mesh,
        out_type=(
            jax.ShapeDtypeStruct((b, h, dim), jnp.bfloat16),
            jax.ShapeDtypeStruct((v, words), jnp.int32),
            jax.ShapeDtypeStruct((v, words), jnp.int32),
        ),
        scratch_types=[
            pltpu.VMEM((CONV_BLK, dim), jnp.bfloat16),
            pltpu.VMEM((CONV_BLK, dim), jnp.bfloat16),
            pltpu.VMEM((CONV_BLK, words), jnp.int32),
            pltpu.VMEM((CONV_BLK, words), jnp.int32),
            pltpu.VMEM((rows_pw, h), jnp.int32),
            pltpu.VMEM((blk_out, words), jnp.int32),
            pltpu.VMEM((blk_out, words), jnp.int32),
            pltpu.VMEM((GPB, h, dim), jnp.bfloat16),
            pltpu.VMEM((GPB, h, dim), jnp.bfloat16),
            pltpu.SemaphoreType.DMA,
            pltpu.SemaphoreType.DMA,
            pltpu.SemaphoreType.DMA,
            pltpu.SemaphoreType.DMA,
            pltpu.SemaphoreType.DMA,
            pltpu.SemaphoreType.DMA,
            pltpu.SemaphoreType.DMA,
            pltpu.SemaphoreType.DMA,
        ],
        compiler_params=pltpu.CompilerParams(
            use_tc_tiling_on_sc=False, needs_layout_passes=False
        ),
    )
    def lookup(w_hbm, idx_hbm, out_hbm, tab0_hbm, tab1_hbm,
               vb0, vb1, vi0, vi1, idxv, rows_v0, rows_v1, vbf0, vbf1,
               cin0, cin1, cout0, cout1, gsem0, gsem1, osem0, osem1):
        c = lax.axis_index("c")
        s = lax.axis_index("s")
        wid = s * num_cores + c
        vbs = [vb0, vb1]
        vis = [vi0, vi1]
        cins = [cin0, cin1]
        couts = [cout0, cout1]

        def convert_phase(tab_hbm):
            base = s * rows_pt

            def w_slice(blkno):
                return w_hbm.at[pl.ds(base + blkno * CONV_BLK, CONV_BLK)]

            def t_slice(blkno):
                return tab_hbm.at[pl.ds(base + blkno * CONV_BLK, CONV_BLK)]

            pltpu.async_copy(w_slice(0), vb0, cin0)
            pltpu.async_copy(w_slice(1), vb1, cin1)

            def cbody(g, carry):
                for k in (0, 1):
                    blkno = 2 * g + k
                    pltpu.make_async_copy(w_slice(0), vbs[k], cins[k]).wait()

                    @pl.when(g >= 1)
                    def _():
                        pltpu.make_async_copy(vis[k], t_slice(0), couts[k]).wait()

                    def rbody(r, carry2):
                        base_r = r * CONV_UNROLL
                        for u in range(CONV_UNROLL):
                            vis[k][base_r + u] = plsc.bitcast(
                                vbs[k][base_r + u], jnp.int32
                            )
                        return carry2

                    lax.fori_loop(0, CONV_BLK // CONV_UNROLL, rbody, 0)
                    pltpu.async_copy(vis[k], t_slice(blkno), couts[k])

                    @pl.when(blkno + 2 < cblocks)
                    def _():
                        pltpu.async_copy(w_slice(blkno + 2), vbs[k], cins[k])

                return carry

            lax.fori_loop(0, cblocks // 2, cbody, 0)
            pltpu.make_async_copy(vi0, t_slice(0), cout0).wait()
            pltpu.make_async_copy(vi1, t_slice(0), cout1).wait()

        def gather_phase(tab_hbm):
            def fire(gb, rows, gsem):
                handles = []
                for j in range(GPB):
                    handles.append(
                        pltpu.async_copy(
                            tab_hbm.at[idxv.at[gb * GPB + j]],
                            rows.at[pl.ds(j * h, h)],
                            gsem,
                        )
                    )
                return handles

            def convert_out(rows, vbuf):
                def obody(i, carry2):
                    base_r = i * h

                    def jbody(j5, carry3):
                        base_j = j5 * 5
                        for u in range(5):
                            vbuf[i, base_j + u] = plsc.bitcast(
                                rows[base_r + base_j + u], jnp.bfloat16
                            )
                        return carry3

                    lax.fori_loop(0, h // 5, jbody, 0)
                    return carry2

                lax.fori_loop(0, GPB, obody, 0)

            rows_bufs = [rows_v0, rows_v1]
            vbf_bufs = [vbf0, vbf1]
            osems = [osem0, osem1]
            gsems = [gsem0, gsem1]

            def gbody(g, carry):
                handles = [fire(2 * g + k, rows_bufs[k], gsems[k]) for k in (0, 1)]
                for k in (0, 1):
                    gb = 2 * g + k
                    for w in handles[k]:
                        w.wait()

                    @pl.when(g >= 1)
                    def _():
                        pltpu.make_async_copy(
                            vbf_bufs[k], out_hbm.at[pl.ds(0, GPB)], osems[k]
                        ).wait()

                    convert_out(rows_bufs[k], vbf_bufs[k])
                    pltpu.async_copy(
                        vbf_bufs[k],
                        out_hbm.at[pl.ds((wid * gblocks + gb) * GPB, GPB)],
                        osems[k],
                    )
                return carry

            lax.fori_loop(0, gblocks // 2, gbody, 0)
            pltpu.make_async_copy(
                vbf0, out_hbm.at[pl.ds(0, GPB)], osem0
            ).wait()
            pltpu.make_async_copy(
                vbf1, out_hbm.at[pl.ds(0, GPB)], osem1
            ).wait()

        @pl.when(c == 0)
        def _():
            convert_phase(tab0_hbm)

        @pl.when(c == 1)
        def _():
            convert_phase(tab1_hbm)

        pltpu.sync_copy(idx_hbm.at[pl.ds(wid * rows_pw, rows_pw)], idxv)
        plsc.subcore_barrier()

        @pl.when(c == 0)
        def _():
            gather_phase(tab0_hbm)

        @pl.when(c == 1)
        def _():
            gather_phase(tab1_hbm)

    return lookup


def kernel(input, weight):
    b, h = input.shape
    v, dim = weight.shape
    info = plsc.get_sparse_core_info()
    out, _, _ = _make_lookup(b, h, v, dim, info.num_cores, info.num_subcores)(
        weight, input.astype(jnp.int32)
    )
    return out
